# baseline (speedup 1.0000x reference)
import functools

import jax
import jax.numpy as jnp
from jax import lax
from jax.experimental import pallas as pl
from jax.experimental.pallas import tpu as pltpu
from jax.experimental.pallas import tpu_sc as plsc

VOCAB = 100000
N_EMBD = 64
BATCH = 16384
SEQ = 200

NC = 2
NS = 16
NW = NC * NS

N_PACK = N_EMBD // 2
B_TOTAL = BATCH * SEQ
B_PER_W = B_TOTAL // NW
NB = 4
HC = 400
N_OUTER = B_PER_W // (NB * HC)


def _emb_kernel(idx_hbm, table_hbm, out_hbm, idx_v, rows_v, isems, gsems,
                wsems):
  wid = lax.axis_index("s") * NC + lax.axis_index("c")
  base = wid * B_PER_W

  def idx_slice(t):
    return idx_hbm.at[pl.ds(pl.multiple_of(base + t * HC, 8), HC)]

  for b in range(NB):
    pltpu.async_copy(idx_slice(b), idx_v.at[b], isems.at[b])

  def body(i, _):
    t0 = NB * i
    gathers = []
    for b in range(NB):
      @pl.when(i > 0)
      def _wb_done():
        pltpu.make_async_copy(
            rows_v.at[b], out_hbm.at[pl.ds(0, HC)], wsems.at[b]
        ).wait()
      pltpu.make_async_copy(idx_slice(0), idx_v.at[b], isems.at[b]).wait()
      gathers.append(pltpu.async_copy(
          table_hbm.at[idx_v.at[b]], rows_v.at[b], gsems.at[b]))
    for b in range(NB):
      gathers[b].wait()
      row0 = base + (t0 + b) * HC
      pltpu.async_copy(rows_v.at[b], out_hbm.at[pl.ds(row0, HC)],
                       wsems.at[b])
      @pl.when(i < N_OUTER - 1)
      def _prefetch():
        pltpu.async_copy(idx_slice(t0 + NB + b), idx_v.at[b], isems.at[b])
    return ()

  lax.fori_loop(0, N_OUTER, body, ())

  for b in range(NB):
    pltpu.make_async_copy(
        rows_v.at[b], out_hbm.at[pl.ds(0, HC)], wsems.at[b]
    ).wait()


@jax.jit
def _embedding_lookup(idx_flat, table_packed):
  mesh = plsc.VectorSubcoreMesh(
      core_axis_name="c", subcore_axis_name="s", num_cores=NC, num_subcores=NS
  )
  f = pl.kernel(
      _emb_kernel,
      out_type=jax.ShapeDtypeStruct((B_TOTAL, N_PACK), jnp.int32),
      mesh=mesh,
      scratch_types=[
          pltpu.VMEM((NB, HC), jnp.int32),
          pltpu.VMEM((NB, HC, N_PACK), jnp.int32),
          pltpu.SemaphoreType.DMA((NB,)),
          pltpu.SemaphoreType.DMA((NB,)),
          pltpu.SemaphoreType.DMA((NB,)),
      ],
      compiler_params=pltpu.CompilerParams(use_tc_tiling_on_sc=False),
  )
  return f(idx_flat, table_packed)


def kernel(idx, table):
  table_packed = jax.lax.bitcast_convert_type(
      table.astype(jnp.bfloat16).reshape(VOCAB, N_PACK, 2), jnp.int32
  ).reshape(VOCAB, N_PACK)
  out = _embedding_lookup(idx.reshape(-1), table_packed)
  out = jax.lax.bitcast_convert_type(out, jnp.bfloat16).reshape(
      B_TOTAL, N_EMBD).astype(jnp.float32)
  return out.reshape(BATCH, SEQ, N_EMBD)


# speedup vs baseline: 2.0288x; 2.0288x over previous
import functools

import jax
import jax.numpy as jnp
from jax import lax
from jax.experimental import pallas as pl
from jax.experimental.pallas import tpu as pltpu
from jax.experimental.pallas import tpu_sc as plsc

VOCAB = 100000
N_EMBD = 64
BATCH = 16384
SEQ = 200

NC = 2
NS = 16
NW = NC * NS

N_PACK = N_EMBD // 2
B_TOTAL = BATCH * SEQ
B_PER_W = B_TOTAL // NW
NB = 4
HC = 400
N_OUTER = B_PER_W // (NB * HC)


def _emb_kernel(idx_hbm, table_hbm, out_hbm, idx_v, rows_v, isems, gsems,
                wsems):
  wid = lax.axis_index("s") * NC + lax.axis_index("c")
  base = wid * B_PER_W

  def idx_slice(t):
    return idx_hbm.at[pl.ds(pl.multiple_of(base + t * HC, 8), HC)]

  for b in range(NB):
    pltpu.async_copy(idx_slice(b), idx_v.at[b], isems.at[b])

  def body(i, _):
    t0 = NB * i
    gathers = []
    for b in range(NB):
      @pl.when(i > 0)
      def _wb_done():
        pltpu.make_async_copy(
            rows_v.at[b], out_hbm.at[pl.ds(0, HC)], wsems.at[b]
        ).wait()
      pltpu.make_async_copy(idx_slice(0), idx_v.at[b], isems.at[b]).wait()
      gathers.append(pltpu.async_copy(
          table_hbm.at[idx_v.at[b]], rows_v.at[b], gsems.at[b]))
    for b in range(NB):
      gathers[b].wait()
      row0 = base + (t0 + b) * HC
      pltpu.async_copy(rows_v.at[b], out_hbm.at[pl.ds(row0, HC)],
                       wsems.at[b])
      @pl.when(i < N_OUTER - 1)
      def _prefetch():
        pltpu.async_copy(idx_slice(t0 + NB + b), idx_v.at[b], isems.at[b])
    return ()

  lax.fori_loop(0, N_OUTER, body, ())

  for b in range(NB):
    pltpu.make_async_copy(
        rows_v.at[b], out_hbm.at[pl.ds(0, HC)], wsems.at[b]
    ).wait()


@jax.jit
def _embedding_lookup(idx_flat, table_packed):
  mesh = plsc.VectorSubcoreMesh(
      core_axis_name="c", subcore_axis_name="s", num_cores=NC, num_subcores=NS
  )
  f = pl.kernel(
      _emb_kernel,
      out_type=jax.ShapeDtypeStruct((B_TOTAL, N_EMBD), jnp.bfloat16),
      mesh=mesh,
      scratch_types=[
          pltpu.VMEM((NB, HC), jnp.int32),
          pltpu.VMEM((NB, HC, N_EMBD), jnp.bfloat16),
          pltpu.SemaphoreType.DMA((NB,)),
          pltpu.SemaphoreType.DMA((NB,)),
          pltpu.SemaphoreType.DMA((NB,)),
      ],
      compiler_params=pltpu.CompilerParams(use_tc_tiling_on_sc=False),
  )
  return f(idx_flat, table_packed)


def kernel(idx, table):
  out = _embedding_lookup(idx.reshape(-1), table.astype(jnp.bfloat16))
  return out.astype(jnp.float32).reshape(BATCH, SEQ, N_EMBD)


# speedup vs baseline: 3.0951x; 1.5256x over previous
import functools

import jax
import jax.numpy as jnp
from jax import lax
from jax.experimental import pallas as pl
from jax.experimental.pallas import tpu as pltpu
from jax.experimental.pallas import tpu_sc as plsc

VOCAB = 100000
N_EMBD = 64
BATCH = 16384
SEQ = 200

NC = 2
NS = 16
NW = NC * NS

B_TOTAL = BATCH * SEQ
B_PER_W = B_TOTAL // NW
NB = 4
HC = 400
N_OUTER = B_PER_W // (NB * HC)


def _emb_kernel(idx_hbm, table_hbm, out_hbm, idx_v, rows_v, isems, gsems,
                wsems):
  wid = lax.axis_index("s") * NC + lax.axis_index("c")
  base = wid * B_PER_W

  def idx_slice(t):
    return idx_hbm.at[pl.ds(pl.multiple_of(base + t * HC, 8), HC)]

  for b in range(NB):
    pltpu.async_copy(idx_slice(b), idx_v.at[b], isems.at[b])

  def body(i, _):
    t0 = NB * i
    gathers = []
    for b in range(NB):
      @pl.when(i > 0)
      def _wb_done():
        pltpu.make_async_copy(
            rows_v.at[b], out_hbm.at[pl.ds(0, HC)], wsems.at[b]
        ).wait()
      pltpu.make_async_copy(idx_slice(0), idx_v.at[b], isems.at[b]).wait()
      gathers.append(pltpu.async_copy(
          table_hbm.at[idx_v.at[b]], rows_v.at[b], gsems.at[b]))
    for b in range(NB):
      gathers[b].wait()
      row0 = base + (t0 + b) * HC
      pltpu.async_copy(rows_v.at[b], out_hbm.at[pl.ds(row0, HC)],
                       wsems.at[b])
      @pl.when(i < N_OUTER - 1)
      def _prefetch():
        pltpu.async_copy(idx_slice(t0 + NB + b), idx_v.at[b], isems.at[b])
    return ()

  lax.fori_loop(0, N_OUTER, body, ())

  for b in range(NB):
    pltpu.make_async_copy(
        rows_v.at[b], out_hbm.at[pl.ds(0, HC)], wsems.at[b]
    ).wait()


@jax.jit
def _embedding_lookup(idx_flat, table):
  mesh = plsc.VectorSubcoreMesh(
      core_axis_name="c", subcore_axis_name="s", num_cores=NC, num_subcores=NS
  )
  f = pl.kernel(
      _emb_kernel,
      out_type=jax.ShapeDtypeStruct((B_TOTAL, N_EMBD), jnp.float32),
      mesh=mesh,
      scratch_types=[
          pltpu.VMEM((NB, HC), jnp.int32),
          pltpu.VMEM((NB, HC, N_EMBD), jnp.float32),
          pltpu.SemaphoreType.DMA((NB,)),
          pltpu.SemaphoreType.DMA((NB,)),
          pltpu.SemaphoreType.DMA((NB,)),
      ],
      compiler_params=pltpu.CompilerParams(use_tc_tiling_on_sc=False),
  )
  return f(idx_flat, table)


def kernel(idx, table):
  out = _embedding_lookup(idx.reshape(-1), table)
  return out.reshape(BATCH, SEQ, N_EMBD)
